# baseline (speedup 1.0000x reference)
import functools

import jax
import jax.numpy as jnp
from jax import lax
from jax.experimental import pallas as pl
from jax.experimental.pallas import tpu as pltpu
from jax.experimental.pallas import tpu_sc as plsc

NC = 2
NS = 16
L = 16
NW = NC * NS
B = 16384
D = 32
BPW = B // NW
CHUNK = 128
NCHUNK = BPW // CHUNK

_mesh = plsc.VectorSubcoreMesh(
    core_axis_name="c", subcore_axis_name="s", num_cores=NC, num_subcores=NS
)


@functools.partial(
    pl.kernel,
    out_type=jax.ShapeDtypeStruct((B,), jnp.float32),
    mesh=_mesh,
    scratch_types=[
        pltpu.VMEM((NCHUNK, CHUNK), jnp.int32),
        pltpu.VMEM((NCHUNK, CHUNK), jnp.int32),
        pltpu.VMEM((BPW, D), jnp.float32),
        pltpu.VMEM((BPW, D), jnp.float32),
        pltpu.VMEM((BPW,), jnp.float32),
        pltpu.SemaphoreType.DMA,
    ],
    compiler_params=pltpu.CompilerParams(
        needs_layout_passes=False, use_tc_tiling_on_sc=False),
)
def _mf_kernel(uids_hbm, iids_hbm, utab_hbm, itab_hbm, out_hbm,
               uid_v, iid_v, urows, irows, out_v, sem):
    wid = lax.axis_index("s") * NC + lax.axis_index("c")

    pltpu.sync_copy(uids_hbm.at[wid], uid_v)
    pltpu.sync_copy(iids_hbm.at[wid], iid_v)

    copies = []
    for j in range(NCHUNK):
        copies.append(pltpu.async_copy(
            utab_hbm.at[uid_v.at[j]], urows.at[pl.ds(j * CHUNK, CHUNK)], sem))
        copies.append(pltpu.async_copy(
            itab_hbm.at[iid_v.at[j]], irows.at[pl.ds(j * CHUNK, CHUNK)], sem))
    for c in copies:
        c.wait()

    iota = lax.iota(jnp.int32, L)

    def blk_body(blk, carry):
        row = blk * L + iota
        acc = jnp.zeros((L,), jnp.float32)
        for d in range(D):
            col = jnp.full((L,), d, jnp.int32)
            acc = acc + (plsc.load_gather(urows, [row, col])
                         * plsc.load_gather(irows, [row, col]))
        out_v[pl.ds(blk * L, L)] = acc
        return carry

    lax.fori_loop(0, BPW // L, blk_body, 0)

    pltpu.sync_copy(out_v, out_hbm.at[pl.ds(wid * BPW, BPW)])


def kernel(user_ids, item_ids, user_table, item_table):
    u = user_ids.astype(jnp.int32).reshape(NW, NCHUNK, CHUNK)
    i = item_ids.astype(jnp.int32).reshape(NW, NCHUNK, CHUNK)
    return _mf_kernel(u, i, user_table, item_table)
